# baseline (speedup 1.0000x reference)
import jax
import jax.numpy as jnp
from jax.experimental import pallas as pl
from jax.experimental.pallas import tpu as pltpu

B = 32
P = 784
GRID = 7
BLK = P // GRID


def _body(s_ref, f_ref, r_ref, o_ref):
    f = f_ref[:, :]
    r = r_ref[:, :]
    m0 = s_ref[0:1, :] == 1
    m1 = s_ref[1:2, :] == 1
    o0 = jnp.where(m0, r, f)
    o1 = jnp.where(m1, r, f)
    o_ref[:, :] = jnp.stack([o0, o1], axis=1).reshape(2 * BLK, B)


@jax.jit
def kernel(reals, fakes, shuffle_indices):
    f2 = fakes.reshape(B, P).transpose(1, 0)
    r2 = reals.reshape(B, P).transpose(1, 0)
    s2 = shuffle_indices.transpose(1, 0)
    out = pl.pallas_call(
        _body,
        grid=(GRID,),
        in_specs=[
            pl.BlockSpec((2, B), lambda i: (0, 0)),
            pl.BlockSpec((BLK, B), lambda i: (i, 0)),
            pl.BlockSpec((BLK, B), lambda i: (i, 0)),
        ],
        out_specs=pl.BlockSpec((2 * BLK, B), lambda i: (i, 0)),
        out_shape=jax.ShapeDtypeStruct((2 * P, B), jnp.float32),
    )(s2, f2, r2)
    return out.reshape(28, 28, 2, B).transpose(3, 0, 1, 2)[:, :, :, :, None]


# speedup vs baseline: 1.7977x; 1.7977x over previous
import jax
import jax.numpy as jnp
from jax.experimental import pallas as pl
from jax.experimental.pallas import tpu as pltpu

B = 32
P = 784


def _body(f_ref, r_ref, s_ref, o_ref):
    f = f_ref[:, :]
    r = r_ref[:, :]
    m0 = s_ref[0:1, :] == 1
    m1 = s_ref[1:2, :] == 1
    o0 = jnp.where(m0, r, f)
    o1 = jnp.where(m1, r, f)
    o_ref[:, :] = jnp.stack([o0, o1], axis=1).reshape(2 * P, B)


@jax.jit
def kernel(reals, fakes, shuffle_indices):
    f2 = fakes.reshape(B, P).transpose(1, 0)
    r2 = reals.reshape(B, P).transpose(1, 0)
    s2 = shuffle_indices.transpose(1, 0)
    out = pl.pallas_call(
        _body,
        out_shape=jax.ShapeDtypeStruct((2 * P, B), jnp.float32),
    )(f2, r2, s2)
    return out.reshape(28, 28, 2, B).transpose(3, 0, 1, 2)[:, :, :, :, None]
